# baseline (speedup 1.0000x reference)
import jax
import jax.numpy as jnp
from jax import lax
from jax.experimental import pallas as pl
from jax.experimental.pallas import tpu as pltpu
from jax.experimental.pallas import tpu_sc as plsc

NUM_USERS = 4000
NUM_ITEMS = 6000
N_NODES = NUM_USERS + NUM_ITEMS
D = 128
RP_ETA = 0.01
RP_ITERATION = 2
BATCH = 1024
N_EDGES = 320000

NC = 2
NS = 16
NW = NC * NS
G = 128
CG = 16
NCHT = 160
E_PAD = NCHT * CG * G
K1 = 7
K0 = NCHT // NS - K1
RPT = 624
R_REM = N_NODES - NS * RPT
B_PER_W = BATCH // NW

_f32 = jnp.float32


def _rp(X, v0):
  XtX = lax.dot_general(X, X, (((0,), (0,)), ((), ())),
                        preferred_element_type=_f32)
  v = v0
  for _ in range(RP_ITERATION):
    v = jnp.dot(XtX, v, preferred_element_type=_f32)
  Xv = jnp.dot(X, v, preferred_element_type=_f32)
  return X - RP_ETA * jnp.dot(Xv, v.T, preferred_element_type=_f32) / jnp.sum(v * v)


def _binhash(X, W):
  return jnp.sign(lax.dot_general(X, W, (((1,), (1,)), ((), ())),
                                  preferred_element_type=_f32))


def _prep_body(u_ref, i_ref, nu_ref, ni_ref, w_ref, a0_ref, e_ref):
  U = _rp(u_ref[...], nu_ref[...])
  I = _rp(i_ref[...], ni_ref[...])
  a0_ref[:NUM_USERS, :] = U
  a0_ref[NUM_USERS:, :] = I
  e_ref[:NUM_ITEMS, 0:D] = _binhash(I, w_ref[...])
  e_ref[NUM_ITEMS:, 0:D] = _binhash(U, w_ref[...])


_prep = pl.pallas_call(
    _prep_body,
    out_shape=[
        jax.ShapeDtypeStruct((N_NODES, D), _f32),
        jax.ShapeDtypeStruct((N_NODES, 3 * D), _f32),
    ],
)


def _sc_layer_body(a_hbm, src_hbm, dst_hbm, w_hbm, p_hbm,
                   acc_sh, src_v, dst_v, w_v, rows0_v, rows1_v, sem0, sem1):
  c = lax.axis_index("c")
  s = lax.axis_index("s")
  wid = c * NS + s
  rows_v = rows0_v

  zero = jnp.zeros((16,), _f32)

  def _zr(r, _):
    for k in range(D // 16):
      rows_v[r, k * 16:(k + 1) * 16] = zero
    return 0

  lax.fori_loop(0, G, _zr, 0)

  n_full = RPT // G
  rem = RPT - n_full * G
  for j in range(n_full):
    pltpu.sync_copy(rows_v, acc_sh.at[pl.ds(s * RPT + j * G, G)])
  if rem:
    pltpu.sync_copy(rows_v.at[pl.ds(0, rem)],
                    acc_sh.at[pl.ds(s * RPT + n_full * G, rem)])

  @pl.when(s == 0)
  def _zero_tail():
    pltpu.sync_copy(rows_v.at[pl.ds(0, R_REM)],
                    acc_sh.at[pl.ds(NS * RPT, R_REM)])

  plsc.subcore_barrier()

  def _scale(rows_ref, g):
    def _j(j, _):
      wvec = w_v[g, pl.ds(j * 16, 16)]
      for l in range(16):
        wv = wvec[l]
        for k in range(D // 16):
          sl = (j * 16 + l, pl.ds(k * 16, 16))
          rows_ref[sl] = rows_ref[sl] * wv
      return 0

    lax.fori_loop(0, G // 16, _j, 0)

  def _chunk(k, _):
    pltpu.sync_copy(src_hbm.at[k], src_v)
    pltpu.sync_copy(dst_hbm.at[k], dst_v)
    pltpu.sync_copy(w_hbm.at[k], w_v)
    pltpu.async_copy(a_hbm.at[src_v.at[0]], rows0_v, sem0)

    def _pair(t, _):
      g0 = 2 * t
      g1 = g0 + 1
      pltpu.async_copy(a_hbm.at[src_v.at[g1]], rows1_v, sem1)
      pltpu.make_async_copy(a_hbm.at[src_v.at[g0]], rows0_v, sem0).wait()
      _scale(rows0_v, g0)
      pltpu.sync_copy(rows0_v, acc_sh.at[dst_v.at[g0]], add=True)

      @pl.when(t < CG // 2 - 1)
      def _prefetch_even():
        pltpu.async_copy(a_hbm.at[src_v.at[g0 + 2]], rows0_v, sem0)

      pltpu.make_async_copy(a_hbm.at[src_v.at[g1]], rows1_v, sem1).wait()
      _scale(rows1_v, g1)
      pltpu.sync_copy(rows1_v, acc_sh.at[dst_v.at[g1]], add=True)
      return 0

    lax.fori_loop(0, CG // 2, _pair, 0)
    return 0

  @pl.when(c == 1)
  def _run_fast():
    lax.fori_loop(s * K1, s * K1 + K1, _chunk, 0)

  @pl.when(c == 0)
  def _run_slow():
    lax.fori_loop(NS * K1 + s * K0, NS * K1 + s * K0 + K0, _chunk, 0)

  plsc.subcore_barrier()

  pltpu.sync_copy(acc_sh.at[pl.ds(s * RPT, RPT)],
                  p_hbm.at[c, pl.ds(s * RPT, RPT)])

  @pl.when(s == 0)
  def _drain_tail():
    pltpu.sync_copy(acc_sh.at[pl.ds(NS * RPT, R_REM)],
                    p_hbm.at[c, pl.ds(NS * RPT, R_REM)])


_sc_layer = pl.kernel(
    _sc_layer_body,
    out_type=jax.ShapeDtypeStruct((NC, N_NODES, D), _f32),
    mesh=plsc.VectorSubcoreMesh(core_axis_name="c", subcore_axis_name="s",
                                num_cores=NC, num_subcores=NS),
    scratch_types=[
        pltpu.VMEM_SHARED((N_NODES, D), _f32),
        pltpu.VMEM((CG, G), jnp.int32),
        pltpu.VMEM((CG, G), jnp.int32),
        pltpu.VMEM((CG, G), _f32),
        pltpu.VMEM((G, D), _f32),
        pltpu.VMEM((G, D), _f32),
        pltpu.SemaphoreType.DMA,
        pltpu.SemaphoreType.DMA,
    ],
)


def _make_comb(col, with_a):
  def body(p_ref, w_ref, e_in_ref, *out_refs):
    del e_in_ref
    A = p_ref[0] + p_ref[1]
    if with_a:
      out_refs[0][...] = A
    e_ref = out_refs[-1]
    e_ref[:NUM_ITEMS, col:col + D] = _binhash(A[NUM_USERS:], w_ref[...])
    e_ref[NUM_ITEMS:, col:col + D] = _binhash(A[:NUM_USERS], w_ref[...])

  out_shape = [jax.ShapeDtypeStruct((N_NODES, 3 * D), _f32)]
  if with_a:
    out_shape.insert(0, jax.ShapeDtypeStruct((N_NODES, D), _f32))
  return pl.pallas_call(
      body,
      out_shape=out_shape,
      input_output_aliases={2: 1 if with_a else 0},
  )


_comb1 = _make_comb(D, True)
_comb2 = _make_comb(2 * D, False)


def _sc_gather_body(e_hbm, uidx_hbm, ug_hbm, idx_v, row_v, sem):
  c = lax.axis_index("c")
  s = lax.axis_index("s")
  wid = c * NS + s
  base = wid * B_PER_W
  pltpu.sync_copy(uidx_hbm.at[pl.ds(base, B_PER_W)], idx_v)
  pltpu.async_copy(e_hbm.at[idx_v], row_v, sem).wait()
  pltpu.sync_copy(row_v, ug_hbm.at[pl.ds(base, B_PER_W)])


_sc_gather = pl.kernel(
    _sc_gather_body,
    out_type=jax.ShapeDtypeStruct((BATCH, 3 * D), _f32),
    mesh=plsc.VectorSubcoreMesh(core_axis_name="c", subcore_axis_name="s",
                                num_cores=NC, num_subcores=NS),
    scratch_types=[
        pltpu.VMEM((B_PER_W,), jnp.int32),
        pltpu.VMEM((B_PER_W, 3 * D), _f32),
        pltpu.SemaphoreType.DMA,
    ],
)


ITEM_PAD = 6144


def _score_body(ug_ref, it_ref, out_ref):
  out_ref[...] = lax.dot_general(ug_ref[...], it_ref[...],
                                 (((1,), (1,)), ((), ())),
                                 preferred_element_type=_f32)


_score = pl.pallas_call(
    _score_body,
    grid=(ITEM_PAD // 1024,),
    in_specs=[
        pl.BlockSpec((BATCH, 3 * D), lambda i: (0, 0)),
        pl.BlockSpec((1024, 3 * D), lambda i: (i, 0)),
    ],
    out_specs=pl.BlockSpec((BATCH, 1024), lambda i: (0, i)),
    out_shape=jax.ShapeDtypeStruct((BATCH, ITEM_PAD), _f32),
)


def kernel(user_index, edge_index, edge_weight, user_table, item_table,
           hash_W, rp_noise_u, rp_noise_i):
  pad = E_PAD - N_EDGES
  zpad_i = jnp.zeros((pad,), edge_index.dtype)
  src = jnp.concatenate([edge_index[0], zpad_i]).reshape(NCHT, CG, G)
  dst = jnp.concatenate([edge_index[1], zpad_i]).reshape(NCHT, CG, G)
  ew = jnp.concatenate([edge_weight, jnp.zeros((pad,), _f32)]
                       ).reshape(NCHT, CG, G)

  A0, E = _prep(user_table, item_table, rp_noise_u, rp_noise_i, hash_W)
  P1 = _sc_layer(A0, src, dst, ew)
  A1, E = _comb1(P1, hash_W, E)
  P2 = _sc_layer(A1, src, dst, ew)
  (E,) = _comb2(P2, hash_W, E)
  UG = _sc_gather(E, user_index + NUM_ITEMS)
  return _score(UG, E)[:, :NUM_ITEMS]


# speedup vs baseline: 1.1358x; 1.1358x over previous
import jax
import jax.numpy as jnp
from jax import lax
from jax.experimental import pallas as pl
from jax.experimental.pallas import tpu as pltpu
from jax.experimental.pallas import tpu_sc as plsc

NUM_USERS = 4000
NUM_ITEMS = 6000
N_NODES = NUM_USERS + NUM_ITEMS
D = 128
RP_ETA = 0.01
RP_ITERATION = 2
BATCH = 1024
N_EDGES = 320000

NC = 2
NS = 16
NW = NC * NS
G = 128
CG = 16
NCHT = 160
E_PAD = NCHT * CG * G
K1 = 7
K0 = NCHT // NS - K1
RPT = 624
R_REM = N_NODES - NS * RPT
B_PER_W = BATCH // NW

_f32 = jnp.float32


def _rp(X, v0):
  XtX = lax.dot_general(X, X, (((0,), (0,)), ((), ())),
                        preferred_element_type=_f32)
  v = v0
  for _ in range(RP_ITERATION):
    v = jnp.dot(XtX, v, preferred_element_type=_f32)
  Xv = jnp.dot(X, v, preferred_element_type=_f32)
  return X - RP_ETA * jnp.dot(Xv, v.T, preferred_element_type=_f32) / jnp.sum(v * v)


def _binhash(X, W):
  return jnp.sign(lax.dot_general(X, W, (((1,), (1,)), ((), ())),
                                  preferred_element_type=_f32))


def _prep_body(u_ref, i_ref, nu_ref, ni_ref, w_ref, a0_ref, b0_ref):
  U = _rp(u_ref[...], nu_ref[...])
  I = _rp(i_ref[...], ni_ref[...])
  a0_ref[:NUM_USERS, :] = U
  a0_ref[NUM_USERS:, :] = I
  b0_ref[:NUM_USERS, :] = _binhash(U, w_ref[...])
  b0_ref[NUM_USERS:, :] = _binhash(I, w_ref[...])


_prep = pl.pallas_call(
    _prep_body,
    out_shape=[
        jax.ShapeDtypeStruct((N_NODES, D), _f32),
        jax.ShapeDtypeStruct((N_NODES, D), _f32),
    ],
)


def _sc_layer_body(a_hbm, src_hbm, dst_hbm, w_hbm, p_hbm,
                   acc_sh, src_v, dst_v, w_v, rows0_v, rows1_v, sem0, sem1):
  c = lax.axis_index("c")
  s = lax.axis_index("s")
  wid = c * NS + s
  rows_v = rows0_v

  zero = jnp.zeros((16,), _f32)

  def _zr(r, _):
    for k in range(D // 16):
      rows_v[r, k * 16:(k + 1) * 16] = zero
    return 0

  lax.fori_loop(0, G, _zr, 0)

  n_full = RPT // G
  rem = RPT - n_full * G
  for j in range(n_full):
    pltpu.sync_copy(rows_v, acc_sh.at[pl.ds(s * RPT + j * G, G)])
  if rem:
    pltpu.sync_copy(rows_v.at[pl.ds(0, rem)],
                    acc_sh.at[pl.ds(s * RPT + n_full * G, rem)])

  @pl.when(s == 0)
  def _zero_tail():
    pltpu.sync_copy(rows_v.at[pl.ds(0, R_REM)],
                    acc_sh.at[pl.ds(NS * RPT, R_REM)])

  plsc.subcore_barrier()

  def _scale(rows_ref, g):
    def _j(j, _):
      wvec = w_v[g, pl.ds(j * 16, 16)]
      for l in range(16):
        wv = wvec[l]
        for k in range(D // 16):
          sl = (j * 16 + l, pl.ds(k * 16, 16))
          rows_ref[sl] = rows_ref[sl] * wv
      return 0

    lax.fori_loop(0, G // 16, _j, 0)

  def _chunk(k, _):
    pltpu.sync_copy(src_hbm.at[k], src_v)
    pltpu.sync_copy(dst_hbm.at[k], dst_v)
    pltpu.sync_copy(w_hbm.at[k], w_v)
    pltpu.async_copy(a_hbm.at[src_v.at[0]], rows0_v, sem0)

    def _pair(t, _):
      g0 = 2 * t
      g1 = g0 + 1
      pltpu.async_copy(a_hbm.at[src_v.at[g1]], rows1_v, sem1)
      pltpu.make_async_copy(a_hbm.at[src_v.at[g0]], rows0_v, sem0).wait()
      _scale(rows0_v, g0)
      pltpu.sync_copy(rows0_v, acc_sh.at[dst_v.at[g0]], add=True)

      @pl.when(t < CG // 2 - 1)
      def _prefetch_even():
        pltpu.async_copy(a_hbm.at[src_v.at[g0 + 2]], rows0_v, sem0)

      pltpu.make_async_copy(a_hbm.at[src_v.at[g1]], rows1_v, sem1).wait()
      _scale(rows1_v, g1)
      pltpu.sync_copy(rows1_v, acc_sh.at[dst_v.at[g1]], add=True)
      return 0

    lax.fori_loop(0, CG // 2, _pair, 0)
    return 0

  @pl.when(c == 1)
  def _run_fast():
    lax.fori_loop(s * K1, s * K1 + K1, _chunk, 0)

  @pl.when(c == 0)
  def _run_slow():
    lax.fori_loop(NS * K1 + s * K0, NS * K1 + s * K0 + K0, _chunk, 0)

  plsc.subcore_barrier()

  pltpu.sync_copy(acc_sh.at[pl.ds(s * RPT, RPT)],
                  p_hbm.at[c, pl.ds(s * RPT, RPT)])

  @pl.when(s == 0)
  def _drain_tail():
    pltpu.sync_copy(acc_sh.at[pl.ds(NS * RPT, R_REM)],
                    p_hbm.at[c, pl.ds(NS * RPT, R_REM)])


_sc_layer = pl.kernel(
    _sc_layer_body,
    out_type=jax.ShapeDtypeStruct((NC, N_NODES, D), _f32),
    mesh=plsc.VectorSubcoreMesh(core_axis_name="c", subcore_axis_name="s",
                                num_cores=NC, num_subcores=NS),
    scratch_types=[
        pltpu.VMEM_SHARED((N_NODES, D), _f32),
        pltpu.VMEM((CG, G), jnp.int32),
        pltpu.VMEM((CG, G), jnp.int32),
        pltpu.VMEM((CG, G), _f32),
        pltpu.VMEM((G, D), _f32),
        pltpu.VMEM((G, D), _f32),
        pltpu.SemaphoreType.DMA,
        pltpu.SemaphoreType.DMA,
    ],
)


def _comb_body(p_ref, w_ref, a_ref, b_ref):
  A = p_ref[0] + p_ref[1]
  a_ref[...] = A
  b_ref[...] = _binhash(A, w_ref[...])


_comb = pl.pallas_call(
    _comb_body,
    out_shape=[
        jax.ShapeDtypeStruct((N_NODES, D), _f32),
        jax.ShapeDtypeStruct((N_NODES, D), _f32),
    ],
)


def _sc_gather_body(b0_hbm, b1_hbm, b2_hbm, uidx_hbm, ug_hbm,
                    idx_v, row_v, sem):
  c = lax.axis_index("c")
  s = lax.axis_index("s")
  wid = c * NS + s
  base = wid * B_PER_W
  pltpu.sync_copy(uidx_hbm.at[pl.ds(base, B_PER_W)], idx_v)
  for t, tab in enumerate((b0_hbm, b1_hbm, b2_hbm)):
    pltpu.async_copy(tab.at[idx_v], row_v, sem).wait()
    pltpu.sync_copy(row_v,
                    ug_hbm.at[pl.ds(base, B_PER_W), pl.ds(t * D, D)])


_sc_gather = pl.kernel(
    _sc_gather_body,
    out_type=jax.ShapeDtypeStruct((BATCH, 3 * D), _f32),
    mesh=plsc.VectorSubcoreMesh(core_axis_name="c", subcore_axis_name="s",
                                num_cores=NC, num_subcores=NS),
    scratch_types=[
        pltpu.VMEM((B_PER_W,), jnp.int32),
        pltpu.VMEM((B_PER_W, D), _f32),
        pltpu.SemaphoreType.DMA,
    ],
)


ITEM_PAD = 6144


def _score_body(ug_ref, it_ref, out_ref):
  out_ref[...] = lax.dot_general(ug_ref[...], it_ref[...],
                                 (((1,), (1,)), ((), ())),
                                 preferred_element_type=_f32)


_score = pl.pallas_call(
    _score_body,
    grid=(ITEM_PAD // 1024,),
    in_specs=[
        pl.BlockSpec((BATCH, 3 * D), lambda i: (0, 0)),
        pl.BlockSpec((1024, 3 * D), lambda i: (i, 0)),
    ],
    out_specs=pl.BlockSpec((BATCH, 1024), lambda i: (0, i)),
    out_shape=jax.ShapeDtypeStruct((BATCH, ITEM_PAD), _f32),
)


def kernel(user_index, edge_index, edge_weight, user_table, item_table,
           hash_W, rp_noise_u, rp_noise_i):
  pad = E_PAD - N_EDGES
  zpad_i = jnp.zeros((pad,), edge_index.dtype)
  src = jnp.concatenate([edge_index[0], zpad_i]).reshape(NCHT, CG, G)
  dst = jnp.concatenate([edge_index[1], zpad_i]).reshape(NCHT, CG, G)
  ew = jnp.concatenate([edge_weight, jnp.zeros((pad,), _f32)]
                       ).reshape(NCHT, CG, G)

  A0, B0 = _prep(user_table, item_table, rp_noise_u, rp_noise_i, hash_W)
  P1 = _sc_layer(A0, src, dst, ew)
  A1, B1 = _comb(P1, hash_W)
  P2 = _sc_layer(A1, src, dst, ew)
  _, B2 = _comb(P2, hash_W)
  UG = _sc_gather(B0, B1, B2, user_index)
  items = jnp.concatenate(
      [B0[NUM_USERS:], B1[NUM_USERS:], B2[NUM_USERS:]], axis=1)
  items = jnp.pad(items, ((0, ITEM_PAD - NUM_ITEMS), (0, 0)))
  return _score(UG, items)[:, :NUM_ITEMS]


# speedup vs baseline: 1.1489x; 1.0115x over previous
import jax
import jax.numpy as jnp
from jax import lax
from jax.experimental import pallas as pl
from jax.experimental.pallas import tpu as pltpu
from jax.experimental.pallas import tpu_sc as plsc

NUM_USERS = 4000
NUM_ITEMS = 6000
N_NODES = NUM_USERS + NUM_ITEMS
D = 128
RP_ETA = 0.01
RP_ITERATION = 2
BATCH = 1024
N_EDGES = 320000

NC = 2
NS = 16
NW = NC * NS
G = 128
CG = 16
NCHT = 160
E_PAD = NCHT * CG * G
K1 = 7
K0 = NCHT // NS - K1
RPT = 624
R_REM = N_NODES - NS * RPT
B_PER_W = BATCH // NW

_f32 = jnp.float32


def _rp(X, v0):
  XtX = lax.dot_general(X, X, (((0,), (0,)), ((), ())),
                        preferred_element_type=_f32)
  v = v0
  for _ in range(RP_ITERATION):
    v = jnp.dot(XtX, v, preferred_element_type=_f32)
  Xv = jnp.dot(X, v, preferred_element_type=_f32)
  return X - RP_ETA * jnp.dot(Xv, v.T, preferred_element_type=_f32) / jnp.sum(v * v)


def _binhash(X, W):
  return jnp.sign(lax.dot_general(X, W, (((1,), (1,)), ((), ())),
                                  preferred_element_type=_f32))


def _prep_body(u_ref, i_ref, nu_ref, ni_ref, w_ref, a0_ref, b0_ref):
  U = _rp(u_ref[...], nu_ref[...])
  I = _rp(i_ref[...], ni_ref[...])
  a0_ref[:NUM_USERS, :] = U
  a0_ref[NUM_USERS:, :] = I
  b0_ref[:NUM_USERS, :] = _binhash(U, w_ref[...])
  b0_ref[NUM_USERS:, :] = _binhash(I, w_ref[...])


_prep = pl.pallas_call(
    _prep_body,
    out_shape=[
        jax.ShapeDtypeStruct((N_NODES, D), _f32),
        jax.ShapeDtypeStruct((N_NODES, D), _f32),
    ],
)


def _sc_layer_body(a_hbm, src_hbm, dst_hbm, w_hbm, p_hbm,
                   acc_sh, src_v, dst_v, w_v, rows0_v, rows1_v,
                   sem0, sem1, sem_st):
  c = lax.axis_index("c")
  s = lax.axis_index("s")
  wid = c * NS + s
  rows_v = rows0_v

  zero = jnp.zeros((16,), _f32)

  def _zr(r, _):
    for k in range(D // 16):
      rows_v[r, k * 16:(k + 1) * 16] = zero
    return 0

  lax.fori_loop(0, G, _zr, 0)

  n_full = RPT // G
  rem = RPT - n_full * G
  for j in range(n_full):
    pltpu.sync_copy(rows_v, acc_sh.at[pl.ds(s * RPT + j * G, G)])
  if rem:
    pltpu.sync_copy(rows_v.at[pl.ds(0, rem)],
                    acc_sh.at[pl.ds(s * RPT + n_full * G, rem)])

  @pl.when(s == 0)
  def _zero_tail():
    pltpu.sync_copy(rows_v.at[pl.ds(0, R_REM)],
                    acc_sh.at[pl.ds(NS * RPT, R_REM)])

  plsc.subcore_barrier()

  def _scale(rows_ref, p, g):
    def _j(j, _):
      wvec = w_v[p, g, pl.ds(j * 16, 16)]
      for l in range(16):
        wv = wvec[l]
        for k in range(D // 16):
          sl = (j * 16 + l, pl.ds(k * 16, 16))
          rows_ref[sl] = rows_ref[sl] * wv
      return 0

    lax.fori_loop(0, G // 16, _j, 0)

  def _stage(k, slot):
    pltpu.async_copy(src_hbm.at[k], src_v.at[slot], sem_st)
    pltpu.async_copy(dst_hbm.at[k], dst_v.at[slot], sem_st)
    pltpu.async_copy(w_hbm.at[k], w_v.at[slot], sem_st)

  def _stage_wait(k, slot):
    pltpu.make_async_copy(src_hbm.at[k], src_v.at[slot], sem_st).wait()
    pltpu.make_async_copy(dst_hbm.at[k], dst_v.at[slot], sem_st).wait()
    pltpu.make_async_copy(w_hbm.at[k], w_v.at[slot], sem_st).wait()

  def _run(base, kcnt):
    _stage(base, 0)

    def _chunk(ch, _):
      p = lax.rem(ch, 2)
      k = base + ch
      _stage_wait(k, p)

      @pl.when(ch < kcnt - 1)
      def _stage_next():
        _stage(k + 1, 1 - p)

      pltpu.async_copy(a_hbm.at[src_v.at[p, 0]], rows0_v, sem0)

      def _pair(t, _):
        g0 = 2 * t
        g1 = g0 + 1
        pltpu.async_copy(a_hbm.at[src_v.at[p, g1]], rows1_v, sem1)
        pltpu.make_async_copy(a_hbm.at[src_v.at[p, g0]], rows0_v, sem0).wait()
        _scale(rows0_v, p, g0)
        pltpu.sync_copy(rows0_v, acc_sh.at[dst_v.at[p, g0]], add=True)

        @pl.when(t < CG // 2 - 1)
        def _prefetch_even():
          pltpu.async_copy(a_hbm.at[src_v.at[p, g0 + 2]], rows0_v, sem0)

        pltpu.make_async_copy(a_hbm.at[src_v.at[p, g1]], rows1_v, sem1).wait()
        _scale(rows1_v, p, g1)
        pltpu.sync_copy(rows1_v, acc_sh.at[dst_v.at[p, g1]], add=True)
        return 0

      lax.fori_loop(0, CG // 2, _pair, 0)
      return 0

    lax.fori_loop(0, kcnt, _chunk, 0)

  @pl.when(c == 1)
  def _run_fast():
    _run(s * K1, K1)

  @pl.when(c == 0)
  def _run_slow():
    _run(NS * K1 + s * K0, K0)

  plsc.subcore_barrier()

  pltpu.sync_copy(acc_sh.at[pl.ds(s * RPT, RPT)],
                  p_hbm.at[c, pl.ds(s * RPT, RPT)])

  @pl.when(s == 0)
  def _drain_tail():
    pltpu.sync_copy(acc_sh.at[pl.ds(NS * RPT, R_REM)],
                    p_hbm.at[c, pl.ds(NS * RPT, R_REM)])


_sc_layer = pl.kernel(
    _sc_layer_body,
    out_type=jax.ShapeDtypeStruct((NC, N_NODES, D), _f32),
    mesh=plsc.VectorSubcoreMesh(core_axis_name="c", subcore_axis_name="s",
                                num_cores=NC, num_subcores=NS),
    scratch_types=[
        pltpu.VMEM_SHARED((N_NODES, D), _f32),
        pltpu.VMEM((2, CG, G), jnp.int32),
        pltpu.VMEM((2, CG, G), jnp.int32),
        pltpu.VMEM((2, CG, G), _f32),
        pltpu.VMEM((G, D), _f32),
        pltpu.VMEM((G, D), _f32),
        pltpu.SemaphoreType.DMA,
        pltpu.SemaphoreType.DMA,
        pltpu.SemaphoreType.DMA,
    ],
)


def _comb_body(p_ref, w_ref, a_ref, b_ref):
  A = p_ref[0] + p_ref[1]
  a_ref[...] = A
  b_ref[...] = _binhash(A, w_ref[...])


_comb = pl.pallas_call(
    _comb_body,
    out_shape=[
        jax.ShapeDtypeStruct((N_NODES, D), _f32),
        jax.ShapeDtypeStruct((N_NODES, D), _f32),
    ],
)


def _sc_gather_body(b0_hbm, b1_hbm, b2_hbm, uidx_hbm, ug_hbm,
                    idx_v, row_v, sem):
  c = lax.axis_index("c")
  s = lax.axis_index("s")
  wid = c * NS + s
  base = wid * B_PER_W
  pltpu.sync_copy(uidx_hbm.at[pl.ds(base, B_PER_W)], idx_v)
  for t, tab in enumerate((b0_hbm, b1_hbm, b2_hbm)):
    pltpu.async_copy(tab.at[idx_v], row_v, sem).wait()
    pltpu.sync_copy(row_v,
                    ug_hbm.at[pl.ds(base, B_PER_W), pl.ds(t * D, D)])


_sc_gather = pl.kernel(
    _sc_gather_body,
    out_type=jax.ShapeDtypeStruct((BATCH, 3 * D), _f32),
    mesh=plsc.VectorSubcoreMesh(core_axis_name="c", subcore_axis_name="s",
                                num_cores=NC, num_subcores=NS),
    scratch_types=[
        pltpu.VMEM((B_PER_W,), jnp.int32),
        pltpu.VMEM((B_PER_W, D), _f32),
        pltpu.SemaphoreType.DMA,
    ],
)


ITEM_PAD = 6144


def _score_body(ug_ref, it_ref, out_ref):
  out_ref[...] = lax.dot_general(ug_ref[...], it_ref[...],
                                 (((1,), (1,)), ((), ())),
                                 preferred_element_type=_f32)


_score = pl.pallas_call(
    _score_body,
    grid=(ITEM_PAD // 1024,),
    in_specs=[
        pl.BlockSpec((BATCH, 3 * D), lambda i: (0, 0)),
        pl.BlockSpec((1024, 3 * D), lambda i: (i, 0)),
    ],
    out_specs=pl.BlockSpec((BATCH, 1024), lambda i: (0, i)),
    out_shape=jax.ShapeDtypeStruct((BATCH, ITEM_PAD), _f32),
)


def kernel(user_index, edge_index, edge_weight, user_table, item_table,
           hash_W, rp_noise_u, rp_noise_i):
  pad = E_PAD - N_EDGES
  zpad_i = jnp.zeros((pad,), edge_index.dtype)
  src = jnp.concatenate([edge_index[0], zpad_i]).reshape(NCHT, CG, G)
  dst = jnp.concatenate([edge_index[1], zpad_i]).reshape(NCHT, CG, G)
  ew = jnp.concatenate([edge_weight, jnp.zeros((pad,), _f32)]
                       ).reshape(NCHT, CG, G)

  A0, B0 = _prep(user_table, item_table, rp_noise_u, rp_noise_i, hash_W)
  P1 = _sc_layer(A0, src, dst, ew)
  A1, B1 = _comb(P1, hash_W)
  P2 = _sc_layer(A1, src, dst, ew)
  _, B2 = _comb(P2, hash_W)
  UG = _sc_gather(B0, B1, B2, user_index)
  items = jnp.concatenate(
      [B0[NUM_USERS:], B1[NUM_USERS:], B2[NUM_USERS:]], axis=1)
  items = jnp.pad(items, ((0, ITEM_PAD - NUM_ITEMS), (0, 0)))
  return _score(UG, items)[:, :NUM_ITEMS]
